# baseline (speedup 1.0000x reference)
import jax
import jax.numpy as jnp
from jax import lax
from jax.experimental import pallas as pl
from jax.experimental.pallas import tpu as pltpu
from jax.experimental.pallas import tpu_sc as plsc

_B = 16384
_NC = 13
_NF = 26
_V = 100000
_D = 64
_T = 1 + _NC + _NF

_info = plsc.get_sparse_core_info()
_NCORE = _info.num_cores
_NSUB = _info.num_subcores
_NW = _NCORE * _NSUB
_ROWS_W = _B // _NW
_R = 32
_NBLK = _ROWS_W // _R
_K = _R * _NF
_GCH = 104
_NG = _K // _GCH


def _body(tab_ref, xcat_ref, xnum_ref, w_ref, b_ref, cls_ref, off_ref,
          out_ref, cat_v, head_v, xcat_v, xnum_v, idx_v, w_v, b_v, cls_v,
          off_v, gsem, osem):
    wid = lax.axis_index("s") * _NCORE + lax.axis_index("c")
    pltpu.sync_copy(w_ref, w_v)
    pltpu.sync_copy(b_ref, b_v)
    pltpu.sync_copy(cls_ref, cls_v)
    pltpu.sync_copy(off_ref, off_v)

    def blk_body(blk, carry):
        base = wid * _ROWS_W + blk * _R
        pltpu.sync_copy(xcat_ref.at[pl.ds(base * _NF, _K)],
                        xcat_v.at[pl.ds(0, _K)])
        pltpu.sync_copy(xnum_ref.at[pl.ds(base * _NC, _R * _NC)],
                        xnum_v.at[pl.ds(0, _R * _NC)])
        for s in range(_K // 16):
            idx_v[pl.ds(s * 16, 16)] = (xcat_v[pl.ds(s * 16, 16)]
                                        + off_v[pl.ds(s * 16, 16)])
        gathers = []
        for g in range(_NG):
            gathers.append(pltpu.async_copy(
                tab_ref.at[idx_v.at[pl.ds(g * _GCH, _GCH)]],
                cat_v.at[pl.ds(g * _GCH, _GCH)], gsem))
        for r in range(_R):
            for c in range(_D // 16):
                head_v[r, 0, pl.ds(c * 16, 16)] = cls_v[pl.ds(c * 16, 16)]
            xr = xnum_v[pl.ds(r * _NC, 16)]
            for i in range(_NC):
                x = xr[i]
                for c in range(_D // 16):
                    head_v[r, 1 + i, pl.ds(c * 16, 16)] = (
                        x * w_v[i, pl.ds(c * 16, 16)]
                        + b_v[i, pl.ds(c * 16, 16)])
        pltpu.sync_copy(head_v, out_ref.at[pl.ds(base, _R), pl.ds(0, 1 + _NC)])
        for cp in gathers:
            cp.wait()
        outs = []
        for r in range(_R):
            outs.append(pltpu.async_copy(
                cat_v.at[pl.ds(r * _NF, _NF)],
                out_ref.at[base + r, pl.ds(1 + _NC, _NF)], osem))
        for cp in outs:
            cp.wait()
        return carry

    lax.fori_loop(0, _NBLK, blk_body, 0)


@jax.jit
def kernel(x_num, x_cat, num_W, num_b, cat_tables, cls_token):
    table = cat_tables.reshape(_NF * _V, _D)
    xcat = x_cat.reshape(_B * _NF).astype(jnp.int32)
    xnum = x_num.reshape(_B * _NC)
    cls = cls_token.reshape(_D)
    off = (jnp.arange(_K, dtype=jnp.int32) % _NF) * _V
    mesh = plsc.VectorSubcoreMesh(core_axis_name="c", subcore_axis_name="s")
    f = pl.kernel(
        _body,
        mesh=mesh,
        compiler_params=pltpu.CompilerParams(use_tc_tiling_on_sc=False),
        out_type=jax.ShapeDtypeStruct((_B, _T, _D), jnp.float32),
        scratch_types=[
            pltpu.VMEM((_K, _D), jnp.float32),
            pltpu.VMEM((_R, 1 + _NC, _D), jnp.float32),
            pltpu.VMEM((_K + 16,), jnp.int32),
            pltpu.VMEM((_R * _NC + 16,), jnp.float32),
            pltpu.VMEM((_K,), jnp.int32),
            pltpu.VMEM((_NC, _D), jnp.float32),
            pltpu.VMEM((_NC, _D), jnp.float32),
            pltpu.VMEM((_D,), jnp.float32),
            pltpu.VMEM((_K,), jnp.int32),
            pltpu.SemaphoreType.DMA,
            pltpu.SemaphoreType.DMA,
        ],
    )
    return f(table, xcat, xnum, num_W, num_b, cls, off)


# speedup vs baseline: 1.0038x; 1.0038x over previous
import jax
import jax.numpy as jnp
from jax import lax
from jax.experimental import pallas as pl
from jax.experimental.pallas import tpu as pltpu
from jax.experimental.pallas import tpu_sc as plsc

_B = 16384
_NC = 13
_NF = 26
_V = 100000
_D = 64
_T = 1 + _NC + _NF

_info = plsc.get_sparse_core_info()
_NCORE = _info.num_cores
_NSUB = _info.num_subcores
_NW = _NCORE * _NSUB
_ROWS_W = _B // _NW
_R = 32
_NBLK = _ROWS_W // _R


def _body(tab_ref, xcatt_ref, xnum_ref, w_ref, b_ref, cls_ref,
          out_ref, cat_v, head_v, idx_v, xnum_v, w_v, b_v, cls_v,
          gsem, osem):
    wid = lax.axis_index("s") * _NCORE + lax.axis_index("c")
    pltpu.sync_copy(w_ref, w_v)
    pltpu.sync_copy(b_ref, b_v)
    pltpu.sync_copy(cls_ref, cls_v)

    def blk_body(blk, carry):
        base = wid * _ROWS_W + blk * _R
        pltpu.sync_copy(xcatt_ref.at[:, pl.ds(base, _R)], idx_v)
        pltpu.sync_copy(xnum_ref.at[pl.ds(base * _NC, _R * _NC)],
                        xnum_v.at[pl.ds(0, _R * _NC)])
        gathers = []
        for f in range(_NF):
            gathers.append(pltpu.async_copy(
                tab_ref.at[f].at[idx_v.at[f]], cat_v.at[f], gsem))
        for r in range(_R):
            for c in range(_D // 16):
                head_v[r, 0, pl.ds(c * 16, 16)] = cls_v[pl.ds(c * 16, 16)]
            xr = xnum_v[pl.ds(r * _NC, 16)]
            for i in range(_NC):
                x = xr[i]
                for c in range(_D // 16):
                    head_v[r, 1 + i, pl.ds(c * 16, 16)] = (
                        x * w_v[i, pl.ds(c * 16, 16)]
                        + b_v[i, pl.ds(c * 16, 16)])
        pltpu.sync_copy(head_v, out_ref.at[pl.ds(base, _R), pl.ds(0, 1 + _NC)])
        for cp in gathers:
            cp.wait()
        outs = []
        for f in range(_NF):
            outs.append(pltpu.async_copy(
                cat_v.at[f], out_ref.at[pl.ds(base, _R), 1 + _NC + f], osem))
        for cp in outs:
            cp.wait()
        return carry

    lax.fori_loop(0, _NBLK, blk_body, 0)


@jax.jit
def kernel(x_num, x_cat, num_W, num_b, cat_tables, cls_token):
    xcatt = x_cat.astype(jnp.int32).T
    xnum = x_num.reshape(_B * _NC)
    cls = cls_token.reshape(_D)
    mesh = plsc.VectorSubcoreMesh(core_axis_name="c", subcore_axis_name="s")
    f = pl.kernel(
        _body,
        mesh=mesh,
        compiler_params=pltpu.CompilerParams(use_tc_tiling_on_sc=False),
        out_type=jax.ShapeDtypeStruct((_B, _T, _D), jnp.float32),
        scratch_types=[
            pltpu.VMEM((_NF, _R, _D), jnp.float32),
            pltpu.VMEM((_R, 1 + _NC, _D), jnp.float32),
            pltpu.VMEM((_NF, _R), jnp.int32),
            pltpu.VMEM((_R * _NC + 16,), jnp.float32),
            pltpu.VMEM((_NC, _D), jnp.float32),
            pltpu.VMEM((_NC, _D), jnp.float32),
            pltpu.VMEM((_D,), jnp.float32),
            pltpu.SemaphoreType.DMA,
            pltpu.SemaphoreType.DMA,
        ],
    )
    return f(cat_tables, xcatt, xnum, num_W, num_b, cls)


# speedup vs baseline: 1.0358x; 1.0319x over previous
import jax
import jax.numpy as jnp
from jax import lax
from jax.experimental import pallas as pl
from jax.experimental.pallas import tpu as pltpu
from jax.experimental.pallas import tpu_sc as plsc

_B = 16384
_NC = 13
_NF = 26
_V = 100000
_D = 64
_T = 1 + _NC + _NF

_info = plsc.get_sparse_core_info()
_NCORE = _info.num_cores
_NSUB = _info.num_subcores
_NW = _NCORE * _NSUB
_ROWS_W = _B // _NW
_R = 16
_NBLK = _ROWS_W // _R


def _body(tabp_ref, xcatt_ref, xnumt_ref, w_ref, b_ref, cls_ref,
          out_ref, row_v, pair_v, idx_v, pofs_v, xcatb_v, xnumb_v,
          w_v, b_v, cls_v, gsem, osem):
    wid = lax.axis_index("s") * _NCORE + lax.axis_index("c")
    wbase = wid * _ROWS_W
    pltpu.sync_copy(w_ref, w_v)
    pltpu.sync_copy(b_ref, b_v)
    pltpu.sync_copy(cls_ref, cls_v)

    def blk_body(blk, carry):
        base = wbase + blk * _R
        qbase = pl.multiple_of(wbase + (blk // 8) * 128, 128)

        @pl.when(blk % 8 == 0)
        def _load_slabs():
            pltpu.sync_copy(xcatt_ref.at[:, pl.ds(qbase, 128)], xcatb_v)
            pltpu.sync_copy(xnumt_ref.at[:, pl.ds(qbase, 128)], xnumb_v)

        col0 = pl.multiple_of((blk % 8) * _R, _R)
        for f in range(_NF):
            v = xcatb_v[f, pl.ds(col0, _R)]
            idx_v[f, pl.ds(0, _R)] = lax.shift_right_logical(v, 1)
            pofs_v[f, pl.ds(0, _R)] = lax.shift_left(
                lax.bitwise_and(v, 1), 6)
        gathers = []
        for f in range(_NF):
            gathers.append(pltpu.async_copy(
                tabp_ref.at[f].at[idx_v.at[f]], pair_v.at[f], gsem))
        @pl.when(blk != 0)
        def _drain_out():
            pltpu.make_async_copy(
                row_v, out_ref.at[pl.ds(base, _R)], osem).wait()
        cls4 = [cls_v[pl.ds(c * 16, 16)] for c in range(_D // 16)]
        for r in range(_R):
            for c in range(_D // 16):
                row_v[r, pl.ds(c * 16, 16)] = cls4[c]
        for i in range(_NC):
            w4 = [w_v[i, pl.ds(c * 16, 16)] for c in range(_D // 16)]
            b4 = [b_v[i, pl.ds(c * 16, 16)] for c in range(_D // 16)]
            xvec = xnumb_v[i, pl.ds(col0, _R)]
            for r in range(_R):
                x = xvec[r]
                for c in range(_D // 16):
                    row_v[r, pl.ds((1 + i) * _D + c * 16, 16)] = (
                        x * w4[c] + b4[c])
        for cp in gathers:
            cp.wait()
        for f in range(_NF):
            pvec = pofs_v[f, pl.ds(0, _R)]
            for r in range(_R):
                ofs = pvec[r]
                for c in range(_D // 16):
                    row_v[r, pl.ds((1 + _NC + f) * _D + c * 16, 16)] = (
                        pair_v[f, r, pl.ds(ofs + c * 16, 16)])
        pltpu.async_copy(row_v, out_ref.at[pl.ds(base, _R)], osem)
        return carry

    lax.fori_loop(0, _NBLK, blk_body, 0)
    pltpu.make_async_copy(
        row_v, out_ref.at[pl.ds(wbase, _R)], osem).wait()


@jax.jit
def kernel(x_num, x_cat, num_W, num_b, cat_tables, cls_token):
    tabp = cat_tables.reshape(_NF, _V // 2, 2 * _D)
    xcatt = x_cat.astype(jnp.int32).T
    xnumt = x_num.T
    cls = cls_token.reshape(_D)
    mesh = plsc.VectorSubcoreMesh(core_axis_name="c", subcore_axis_name="s")
    f = pl.kernel(
        _body,
        mesh=mesh,
        out_type=jax.ShapeDtypeStruct((_B, _T * _D), jnp.float32),
        scratch_types=[
            pltpu.VMEM((_R, _T * _D), jnp.float32),
            pltpu.VMEM((_NF, _R, 2 * _D), jnp.float32),
            pltpu.VMEM((_NF, _R), jnp.int32),
            pltpu.VMEM((_NF, _R), jnp.int32),
            pltpu.VMEM((_NF, 128), jnp.int32),
            pltpu.VMEM((_NC, 128), jnp.float32),
            pltpu.VMEM((_NC, _D), jnp.float32),
            pltpu.VMEM((_NC, _D), jnp.float32),
            pltpu.VMEM((_D,), jnp.float32),
            pltpu.SemaphoreType.DMA,
            pltpu.SemaphoreType.DMA,
        ],
    )
    return f(tabp, xcatt, xnumt, num_W, num_b, cls).reshape(_B, _T, _D)
